# baseline (speedup 1.0000x reference)
import functools

import jax
import jax.numpy as jnp
from jax import lax
from jax.experimental import pallas as pl
from jax.experimental.layout import Format, Layout
from jax.experimental.pallas import tpu as pltpu
from jax.experimental.pallas import tpu_sc as plsc

_INFO = plsc.get_sparse_core_info()
_NC = _INFO.num_cores
_NS = _INFO.num_subcores
_NW = _NC * _NS

_NBUF = 4


@functools.cache
def _build(b: int, h: int, vocab: int, d: int):
    n = b * h
    bpw = n // _NW
    rpw = b // _NW
    c = h
    ng = rpw

    mesh = plsc.VectorSubcoreMesh(core_axis_name="c", subcore_axis_name="s")

    @functools.partial(
        pl.kernel,
        mesh=mesh,
        out_type=jax.ShapeDtypeStruct((b, h, d), jnp.float32),
        scratch_types=[
            pltpu.VMEM((bpw,), jnp.int32),
            *[pltpu.VMEM((c, d), jnp.float32) for _ in range(_NBUF)],
            *[pltpu.SemaphoreType.DMA for _ in range(2 * _NBUF)],
        ],
        compiler_params=pltpu.CompilerParams(use_tc_tiling_on_sc=False),
    )
    def emb(ids_hbm, table_hbm, out_hbm, idx_v, *bufs):
        rows = bufs[:_NBUF]
        sg = bufs[_NBUF:2 * _NBUF]
        so = bufs[2 * _NBUF:]
        wid = lax.axis_index("s") * _NC + lax.axis_index("c")
        base = wid * bpw
        row0 = wid * rpw
        pltpu.sync_copy(ids_hbm.at[pl.ds(base, bpw)], idx_v)

        def gather(gi, bi):
            return pltpu.make_async_copy(
                table_hbm.at[idx_v.at[pl.ds(gi * c, c)]], rows[bi], sg[bi])

        def write(gi, bi):
            return pltpu.make_async_copy(
                rows[bi], out_hbm.at[row0 + gi], so[bi])

        for bi in range(_NBUF):
            gather(bi, bi).start()

        def outer(i, carry):
            for bi in range(_NBUF):
                g = i * _NBUF + bi
                gather(g, bi).wait()
                write(g, bi).start()
                write(g, bi).wait()
                gather(g + _NBUF, bi).start()
            return carry

        lax.fori_loop(0, ng // _NBUF - 1, outer, 0)

        for bi in range(_NBUF):
            g = ng - _NBUF + bi
            gather(g, bi).wait()
            write(g, bi).start()
        for bi in range(_NBUF):
            g = ng - _NBUF + bi
            write(g, bi).wait()

    return emb


def kernel(input_ids, table):
    b, h = input_ids.shape
    vocab, d = table.shape
    ids = input_ids.reshape(-1).astype(jnp.int32)
    one = (ids[0] * 0 + 1).astype(jnp.float32)
    t_lin = table * one
    return _build(b, h, vocab, d)(ids, t_lin)


# speedup vs baseline: 1.0072x; 1.0072x over previous
import functools

import jax
import jax.numpy as jnp
from jax import lax
from jax.experimental import pallas as pl
from jax.experimental.pallas import tpu as pltpu
from jax.experimental.pallas import tpu_sc as plsc

_INFO = plsc.get_sparse_core_info()
_NC = _INFO.num_cores
_NS = _INFO.num_subcores
_NW = _NC * _NS


@functools.cache
def _build(b: int, h: int, vocab: int, d: int):
    bpw = b // _NW

    mesh = plsc.VectorSubcoreMesh(core_axis_name="c", subcore_axis_name="s")

    @functools.partial(
        pl.kernel,
        mesh=mesh,
        out_type=jax.ShapeDtypeStruct((h, b, d), jnp.float32),
        scratch_types=[
            pltpu.VMEM((h, bpw), jnp.int32),
            pltpu.VMEM((bpw, d), jnp.float32),
            pltpu.VMEM((bpw, d), jnp.float32),
            *[pltpu.SemaphoreType.DMA for _ in range(4)],
        ],
        compiler_params=pltpu.CompilerParams(use_tc_tiling_on_sc=False),
    )
    def emb(ids_hbm, table_hbm, out_hbm, idx_v, r0, r1, *sems):
        rows = (r0, r1)
        sg = sems[:2]
        so = sems[2:]
        wid = lax.axis_index("s") * _NC + lax.axis_index("c")
        b0 = wid * bpw
        pltpu.sync_copy(ids_hbm.at[:, pl.ds(b0, bpw)], idx_v)

        def gather(hh, bi):
            return pltpu.make_async_copy(
                table_hbm.at[idx_v.at[hh]], rows[bi], sg[bi])

        def write(hh, bi):
            return pltpu.make_async_copy(
                rows[bi], out_hbm.at[hh, pl.ds(b0, bpw)], so[bi])

        for bi in range(2):
            gather(bi, bi).start()

        def outer(i, carry):
            for bi in range(2):
                hh = i * 2 + bi
                gather(hh, bi).wait()
                write(hh, bi).start()
                write(hh, bi).wait()
                gather(hh + 2, bi).start()
            return carry

        lax.fori_loop(0, h // 2 - 1, outer, 0)

        for bi in range(2):
            hh = h - 2 + bi
            gather(hh, bi).wait()
            write(hh, bi).start()
        for bi in range(2):
            write(h - 2 + bi, bi).wait()

    return emb


def kernel(input_ids, table):
    b, h = input_ids.shape
    vocab, d = table.shape
    ids_t = input_ids.T.astype(jnp.int32)
    out_t = _build(b, h, vocab, d)(ids_t, table)
    return out_t.transpose(1, 0, 2)


# speedup vs baseline: 1.0452x; 1.0377x over previous
import functools

import jax
import jax.numpy as jnp
from jax import lax
from jax.experimental import pallas as pl
from jax.experimental.pallas import tpu as pltpu
from jax.experimental.pallas import tpu_sc as plsc

_INFO = plsc.get_sparse_core_info()
_NC = _INFO.num_cores
_NS = _INFO.num_subcores
_NW = _NC * _NS


@functools.cache
def _build(b: int, h: int, vocab: int, d: int):
    bpw = b // _NW

    mesh = plsc.VectorSubcoreMesh(core_axis_name="c", subcore_axis_name="s")

    @functools.partial(
        pl.kernel,
        mesh=mesh,
        out_type=jax.ShapeDtypeStruct((h, b, 2 * d), jnp.float32),
        scratch_types=[
            pltpu.VMEM((h, bpw), jnp.int32),
            pltpu.VMEM((bpw, d), jnp.float32),
            pltpu.VMEM((bpw, d), jnp.float32),
            *[pltpu.SemaphoreType.DMA for _ in range(4)],
        ],
        compiler_params=pltpu.CompilerParams(use_tc_tiling_on_sc=False),
    )
    def emb(ids_hbm, table_hbm, out_hbm, idx_v, r0, r1, *sems):
        rows = (r0, r1)
        sg = sems[:2]
        so = sems[2:]
        wid = lax.axis_index("s") * _NC + lax.axis_index("c")
        b0 = wid * bpw
        pltpu.sync_copy(ids_hbm.at[:, pl.ds(b0, bpw)], idx_v)

        def gather(hh, bi):
            return pltpu.make_async_copy(
                table_hbm.at[idx_v.at[hh]], rows[bi], sg[bi])

        def write(hh, bi):
            return pltpu.make_async_copy(
                rows[bi], out_hbm.at[hh, pl.ds(b0, bpw), pl.ds(0, d)],
                so[bi])

        for bi in range(2):
            gather(bi, bi).start()

        def outer(i, carry):
            for bi in range(2):
                hh = i * 2 + bi
                gather(hh, bi).wait()
                write(hh, bi).start()
                write(hh, bi).wait()
                gather(hh + 2, bi).start()
            return carry

        lax.fori_loop(0, h // 2 - 1, outer, 0)

        for bi in range(2):
            hh = h - 2 + bi
            gather(hh, bi).wait()
            write(hh, bi).start()
        for bi in range(2):
            write(h - 2 + bi, bi).wait()

    return emb


def kernel(input_ids, table):
    b, h = input_ids.shape
    vocab, d = table.shape
    ids_t = input_ids.T.astype(jnp.int32)
    out_t = _build(b, h, vocab, d)(ids_t, table)
    return out_t[:, :, :d].transpose(1, 0, 2)


# speedup vs baseline: 1.0474x; 1.0021x over previous
import functools

import jax
import jax.numpy as jnp
from jax import lax
from jax.experimental import pallas as pl
from jax.experimental.pallas import tpu as pltpu
from jax.experimental.pallas import tpu_sc as plsc

_INFO = plsc.get_sparse_core_info()
_NC = _INFO.num_cores
_NS = _INFO.num_subcores
_NW = _NC * _NS


@functools.cache
def _build(b: int, h: int, vocab: int, d: int):
    bpw = b // _NW

    mesh = plsc.VectorSubcoreMesh(core_axis_name="c", subcore_axis_name="s")

    @functools.partial(
        pl.kernel,
        mesh=mesh,
        out_type=jax.ShapeDtypeStruct((h, b, 2 * d), jnp.float32),
        scratch_types=[
            pltpu.VMEM((h, bpw), jnp.int32),
            pltpu.VMEM((bpw, d), jnp.float32),
            pltpu.VMEM((bpw, d), jnp.float32),
            *[pltpu.SemaphoreType.DMA for _ in range(4)],
        ],
        compiler_params=pltpu.CompilerParams(use_tc_tiling_on_sc=False),
    )
    def emb(ids_hbm, table_hbm, out_hbm, idx_v, r0, r1, *sems):
        rows = (r0, r1)
        sg = sems[:2]
        so = sems[2:]
        wid = lax.axis_index("s") * _NC + lax.axis_index("c")
        b0 = wid * bpw
        pltpu.sync_copy(ids_hbm.at[:, pl.ds(b0, bpw)], idx_v)

        def gather(hh, bi):
            return pltpu.make_async_copy(
                table_hbm.at[idx_v.at[hh]], rows[bi], sg[bi])

        def write(hh, bi):
            return pltpu.make_async_copy(
                rows[bi], out_hbm.at[hh, pl.ds(b0, bpw), pl.ds(0, d)],
                so[bi])

        for bi in range(2):
            gather(bi, bi).start()

        def outer(i, carry):
            for bi in range(2):
                hh = i * 2 + bi
                gather(hh, bi).wait()
                write(hh, bi).start()
                write(hh, bi).wait()
                gather(hh + 2, bi).start()
            return carry

        lax.fori_loop(0, h // 2 - 1, outer, 0)

        for bi in range(2):
            hh = h - 2 + bi
            gather(hh, bi).wait()
            write(hh, bi).start()
        for bi in range(2):
            write(h - 2 + bi, bi).wait()

    return emb


def kernel(input_ids, table):
    b, h = input_ids.shape
    vocab, d = table.shape
    ids_t = input_ids.T.astype(jnp.int32)
    t_lin = lax.optimization_barrier(
        table.reshape(vocab // 2, 2 * d)).reshape(vocab, d)
    out_t = _build(b, h, vocab, d)(ids_t, t_lin)
    return out_t[:, :, :d].transpose(1, 0, 2)
